# Initial kernel scaffold; baseline (speedup 1.0000x reference)
import jax
import jax.numpy as jnp
from jax.experimental import pallas as pl


def kernel(batch_attributes, batch_values, proto_bank):
    raise NotImplementedError("write your pallas kernel here")


# speedup vs baseline: 2.1301x; 2.1301x over previous
import functools

import jax
import jax.numpy as jnp
from jax import lax
from jax.experimental import pallas as pl
from jax.experimental.pallas import tpu as pltpu
from jax.experimental.pallas import tpu_sc as plsc

A = 2000
V = 50
D = 128
B = 4096
PAD = 56


def _neg_idx_body(a_ref, v_ref, o_ref):
    j = lax.broadcasted_iota(jnp.int32, (B, PAD), 1)
    a = a_ref[...]
    v = v_ref[...]
    neg = a * V + j + (j >= v).astype(jnp.int32)
    pos = a * V + v
    o_ref[...] = jnp.where(j < V - 1, neg, pos)


def kernel(batch_attributes, batch_values, proto_bank):
    table = proto_bank.reshape(A * V, D)

    idx2d = pl.pallas_call(
        _neg_idx_body,
        out_shape=jax.ShapeDtypeStruct((B, PAD), jnp.int32),
    )(batch_attributes.reshape(B, 1), batch_values.reshape(B, 1))
    nidx_flat = idx2d.reshape(B * PAD)

    info = plsc.get_sparse_core_info()
    NC, NS, L = info.num_cores, info.num_subcores, info.num_lanes
    NW = NC * NS
    S = B // NW
    G = 2 * PAD
    NG = S // 2

    mesh = plsc.VectorSubcoreMesh(core_axis_name="c", subcore_axis_name="s")

    @functools.partial(
        pl.kernel,
        mesh=mesh,
        out_type=(
            jax.ShapeDtypeStruct((B, D), jnp.float32),
            jax.ShapeDtypeStruct((B, V - 1, D), jnp.float32),
        ),
        scratch_types=[
            pltpu.VMEM((S,), jnp.int32),
            pltpu.VMEM((S,), jnp.int32),
            pltpu.VMEM((S,), jnp.int32),
            pltpu.VMEM((S * PAD,), jnp.int32),
            pltpu.VMEM((S, D), jnp.float32),
            pltpu.VMEM((G, D), jnp.float32),
            pltpu.VMEM((G, D), jnp.float32),
            pltpu.SemaphoreType.DMA,
            pltpu.SemaphoreType.DMA,
            pltpu.SemaphoreType.DMA,
        ],
    )
    def sc_kernel(attr_hbm, val_hbm, nidx_hbm, table_hbm, pos_hbm, neg_hbm,
                  attr_v, val_v, pidx_v, nidx_v, posbuf, nbuf0, nbuf1,
                  psem, sem0, sem1):
        wid = lax.axis_index("s") * NC + lax.axis_index("c")
        base = wid * S
        pltpu.sync_copy(attr_hbm.at[pl.ds(base, S)], attr_v)
        pltpu.sync_copy(val_hbm.at[pl.ds(base, S)], val_v)
        pltpu.sync_copy(nidx_hbm.at[pl.ds(base * PAD, S * PAD)], nidx_v)

        for c in range(S // L):
            s0 = c * L
            a = attr_v[pl.ds(s0, L)]
            v = val_v[pl.ds(s0, L)]
            pidx_v[pl.ds(s0, L)] = a * V + v

        pos_copy = pltpu.async_copy(table_hbm.at[pidx_v], posbuf, psem)

        def gather(g, buf, sem):
            return pltpu.async_copy(
                table_hbm.at[nidx_v.at[pl.ds(g * G, G)]], buf, sem)

        def drain(g, buf):
            s0 = base + 2 * g
            pltpu.sync_copy(buf.at[pl.ds(0, V - 1)], neg_hbm.at[s0])
            pltpu.sync_copy(buf.at[pl.ds(PAD, V - 1)], neg_hbm.at[s0 + 1])

        gather(0, nbuf0, sem0)

        def step(i, carry):
            g0 = 2 * i
            gather(g0 + 1, nbuf1, sem1)
            pltpu.make_async_copy(
                table_hbm.at[nidx_v.at[pl.ds(g0 * G, G)]], nbuf0, sem0).wait()
            drain(g0, nbuf0)

            @pl.when(g0 + 2 < NG)
            def _():
                gather(g0 + 2, nbuf0, sem0)

            pltpu.make_async_copy(
                table_hbm.at[nidx_v.at[pl.ds((g0 + 1) * G, G)]], nbuf1,
                sem1).wait()
            drain(g0 + 1, nbuf1)
            return carry

        lax.fori_loop(0, NG // 2, step, 0)

        pos_copy.wait()
        pltpu.sync_copy(posbuf, pos_hbm.at[pl.ds(base, S)])

    pos, neg = sc_kernel(batch_attributes, batch_values, nidx_flat, table)
    return pos, neg


# speedup vs baseline: 2.2468x; 1.0548x over previous
import functools

import jax
import jax.numpy as jnp
from jax import lax
from jax.experimental import pallas as pl
from jax.experimental.pallas import tpu as pltpu
from jax.experimental.pallas import tpu_sc as plsc

A = 2000
V = 50
D = 128
B = 4096
PAD = 56


def _neg_idx_body(a_ref, v_ref, o_ref):
    j = lax.broadcasted_iota(jnp.int32, (B, PAD), 1)
    a = a_ref[...]
    v = v_ref[...]
    neg = a * V + j + (j >= v).astype(jnp.int32)
    pos = a * V + v
    o_ref[...] = jnp.where(j < V - 1, neg, pos)


def kernel(batch_attributes, batch_values, proto_bank):
    table = proto_bank.reshape(A * V, D)

    idx2d = pl.pallas_call(
        _neg_idx_body,
        out_shape=jax.ShapeDtypeStruct((B, PAD), jnp.int32),
    )(batch_attributes.reshape(B, 1), batch_values.reshape(B, 1))
    nidx_flat = idx2d.reshape(B * PAD)

    info = plsc.get_sparse_core_info()
    NC, NS, L = info.num_cores, info.num_subcores, info.num_lanes
    NW = NC * NS
    S = B // NW
    G = 2 * PAD
    NG = S // 2

    mesh = plsc.VectorSubcoreMesh(core_axis_name="c", subcore_axis_name="s")

    @functools.partial(
        pl.kernel,
        mesh=mesh,
        out_type=(
            jax.ShapeDtypeStruct((B, D), jnp.float32),
            jax.ShapeDtypeStruct((B, V - 1, D), jnp.float32),
        ),
        scratch_types=[
            pltpu.VMEM((S,), jnp.int32),
            pltpu.VMEM((S,), jnp.int32),
            pltpu.VMEM((S,), jnp.int32),
            pltpu.VMEM((S * PAD,), jnp.int32),
            pltpu.VMEM((S, D), jnp.float32),
            pltpu.VMEM((4, G, D), jnp.float32),
            pltpu.SemaphoreType.DMA,
            pltpu.SemaphoreType.DMA,
            pltpu.SemaphoreType.DMA,
            pltpu.SemaphoreType.DMA,
            pltpu.SemaphoreType.DMA,
            pltpu.SemaphoreType.DMA,
            pltpu.SemaphoreType.DMA,
            pltpu.SemaphoreType.DMA,
            pltpu.SemaphoreType.DMA,
        ],
    )
    def sc_kernel(attr_hbm, val_hbm, nidx_hbm, table_hbm, pos_hbm, neg_hbm,
                  attr_v, val_v, pidx_v, nidx_v, posbuf, nring,
                  psem, g0s, g1s, g2s, g3s, d0s, d1s, d2s, d3s):
        gsems = (g0s, g1s, g2s, g3s)
        dsems = (d0s, d1s, d2s, d3s)
        wid = lax.axis_index("s") * NC + lax.axis_index("c")
        base = wid * S
        pltpu.sync_copy(attr_hbm.at[pl.ds(base, S)], attr_v)
        pltpu.sync_copy(val_hbm.at[pl.ds(base, S)], val_v)
        pltpu.sync_copy(nidx_hbm.at[pl.ds(base * PAD, S * PAD)], nidx_v)

        for c in range(S // L):
            s0 = c * L
            a = attr_v[pl.ds(s0, L)]
            v = val_v[pl.ds(s0, L)]
            pidx_v[pl.ds(s0, L)] = a * V + v

        pos_copy = pltpu.async_copy(table_hbm.at[pidx_v], posbuf, psem)

        def gather(g, b):
            pltpu.async_copy(
                table_hbm.at[nidx_v.at[pl.ds(g * G, G)]], nring.at[b],
                gsems[b])

        def wait_gather(g, b):
            pltpu.make_async_copy(
                table_hbm.at[nidx_v.at[pl.ds(g * G, G)]], nring.at[b],
                gsems[b]).wait()

        def drain(g, b):
            s0 = base + 2 * g
            buf = nring.at[b]
            pltpu.async_copy(buf.at[pl.ds(0, V - 1)], neg_hbm.at[s0],
                             dsems[b])
            pltpu.async_copy(buf.at[pl.ds(PAD, V - 1)], neg_hbm.at[s0 + 1],
                             dsems[b])

        def wait_drain(g, b):
            s0 = base + 2 * g
            buf = nring.at[b]
            pltpu.make_async_copy(buf.at[pl.ds(0, V - 1)], neg_hbm.at[s0],
                                  dsems[b]).wait()
            pltpu.make_async_copy(buf.at[pl.ds(PAD, V - 1)],
                                  neg_hbm.at[s0 + 1], dsems[b]).wait()

        gather(0, 0)
        gather(1, 1)
        gather(2, 2)

        def step(r, carry):
            for k in range(4):
                b = k
                g = 4 * r + k
                bn = (k + 3) % 4
                wait_gather(g, b)
                drain(g, b)
                if k == 0:
                    @pl.when(r >= 1)
                    def _():
                        wait_drain(g - 1, bn)

                    gather(g + 3, bn)
                else:
                    wait_drain(g - 1, bn)

                    @pl.when(g + 3 < NG)
                    def _():
                        gather(g + 3, bn)
            return carry

        lax.fori_loop(0, NG // 4, step, 0)

        wait_drain(NG - 1, 3)
        pos_copy.wait()
        pltpu.sync_copy(posbuf, pos_hbm.at[pl.ds(base, S)])

    pos, neg = sc_kernel(batch_attributes, batch_values, nidx_flat, table)
    return pos, neg


# speedup vs baseline: 6.4611x; 2.8757x over previous
import functools

import jax
import jax.numpy as jnp
from jax import lax
from jax.experimental import pallas as pl
from jax.experimental.pallas import tpu as pltpu
from jax.experimental.pallas import tpu_sc as plsc

A = 2000
V = 50
D = 128
B = 4096
NJ = V - 1
NBUF = 6


def kernel(batch_attributes, batch_values, proto_bank):
    table = proto_bank.transpose(1, 0, 2).reshape(V * A, D)

    info = plsc.get_sparse_core_info()
    NC, NS, L = info.num_cores, info.num_subcores, info.num_lanes
    NW = NC * NS
    S = B // NW

    mesh = plsc.VectorSubcoreMesh(core_axis_name="c", subcore_axis_name="s")

    @functools.partial(
        pl.kernel,
        mesh=mesh,
        out_type=(
            jax.ShapeDtypeStruct((B, D), jnp.float32),
            jax.ShapeDtypeStruct((NJ * B, D), jnp.float32),
        ),
        scratch_types=[
            pltpu.VMEM((S,), jnp.int32),
            pltpu.VMEM((S,), jnp.int32),
            pltpu.VMEM((S,), jnp.int32),
            pltpu.VMEM((NJ, S), jnp.int32),
            pltpu.VMEM((S, D), jnp.float32),
            pltpu.VMEM((NBUF, S, D), jnp.float32),
            pltpu.SemaphoreType.DMA,
            pltpu.SemaphoreType.DMA,
            pltpu.SemaphoreType.DMA,
            pltpu.SemaphoreType.DMA,
            pltpu.SemaphoreType.DMA,
            pltpu.SemaphoreType.DMA,
            pltpu.SemaphoreType.DMA,
            pltpu.SemaphoreType.DMA,
            pltpu.SemaphoreType.DMA,
            pltpu.SemaphoreType.DMA,
            pltpu.SemaphoreType.DMA,
            pltpu.SemaphoreType.DMA,
            pltpu.SemaphoreType.DMA,
        ],
    )
    def sc_kernel(attr_hbm, val_hbm, table_hbm, pos_hbm, neg_hbm,
                  attr_v, val_v, pidx_v, nidx_v, posbuf, ring,
                  psem, g0s, g1s, g2s, g3s, g4s, g5s,
                  d0s, d1s, d2s, d3s, d4s, d5s):
        gsems = (g0s, g1s, g2s, g3s, g4s, g5s)
        dsems = (d0s, d1s, d2s, d3s, d4s, d5s)
        wid = lax.axis_index("s") * NC + lax.axis_index("c")
        base = wid * S
        pltpu.sync_copy(attr_hbm.at[pl.ds(base, S)], attr_v)
        pltpu.sync_copy(val_hbm.at[pl.ds(base, S)], val_v)

        def compute_planes(lo, hi):
            def jloop(j, carry):
                for c in range(S // L):
                    s0 = c * L
                    a = attr_v[pl.ds(s0, L)]
                    v = val_v[pl.ds(s0, L)]
                    nidx_v[j, pl.ds(s0, L)] = (
                        j * A + a + jnp.where(v <= j, A, 0))
                return carry

            lax.fori_loop(lo, hi, jloop, 0)

        def gather(j, b):
            pltpu.async_copy(table_hbm.at[nidx_v.at[j]], ring.at[b], gsems[b])

        def wait_gather(j, b):
            pltpu.make_async_copy(
                table_hbm.at[nidx_v.at[j]], ring.at[b], gsems[b]).wait()

        def drain(j, b):
            pltpu.async_copy(
                ring.at[b], neg_hbm.at[pl.ds(j * B + base, S)], dsems[b])

        def wait_drain(j, b):
            pltpu.make_async_copy(
                ring.at[b], neg_hbm.at[pl.ds(j * B + base, S)],
                dsems[b]).wait()

        compute_planes(0, NBUF - 1)
        for b in range(NBUF - 1):
            gather(b, b)

        for c in range(S // L):
            s0 = c * L
            a = attr_v[pl.ds(s0, L)]
            v = val_v[pl.ds(s0, L)]
            pidx_v[pl.ds(s0, L)] = v * A + a
        pos_copy = pltpu.async_copy(table_hbm.at[pidx_v], posbuf, psem)
        compute_planes(NBUF - 1, NJ)

        def step(r, carry):
            for k in range(NBUF):
                b = k
                j = NBUF * r + k
                bn = (k + NBUF - 1) % NBUF
                wait_gather(j, b)
                drain(j, b)
                if k == 0:
                    @pl.when(r >= 1)
                    def _():
                        wait_drain(j - 1, bn)

                    gather(j + NBUF - 1, bn)
                else:
                    wait_drain(j - 1, bn)

                    @pl.when(j + NBUF - 1 < NJ)
                    def _():
                        gather(j + NBUF - 1, bn)
            return carry

        lax.fori_loop(0, NJ // NBUF, step, 0)

        wait_drain(NJ - 2, (NJ - 2) % NBUF)
        wait_gather(NJ - 1, (NJ - 1) % NBUF)
        drain(NJ - 1, (NJ - 1) % NBUF)
        pos_copy.wait()
        pltpu.sync_copy(posbuf, pos_hbm.at[pl.ds(base, S)])
        wait_drain(NJ - 1, (NJ - 1) % NBUF)

    pos, neg = sc_kernel(batch_attributes, batch_values, table)
    return pos, neg.reshape(NJ, B, D).transpose(1, 0, 2)
